# Initial kernel scaffold; baseline (speedup 1.0000x reference)
import functools
import math

import jax
import jax.numpy as jnp
from jax import lax
from jax.experimental import pallas as pl
from jax.experimental.pallas import tpu as pltpu


def _proj_body(xq, xk, xv, wq, wk, wv, bq, bk, bv, qo, ko, vo):
    qo[...] = jnp.dot(xq[...], wq[...], preferred_element_type=jnp.float32) + bq[...]
    ko[...] = jnp.dot(xk[...], wk[...], preferred_element_type=jnp.float32) + bk[...]
    vo[...] = jnp.dot(xv[...], wv[...], preferred_element_type=jnp.float32) + bv[...]


def _projections(xq, xk, xv, Wq, bq, Wk, bk, Wv, bv):
    M, D = xq.shape
    TM = 512
    grid = (M // TM,)
    row = lambda i: (i, 0)
    fixed = lambda i: (0, 0)
    out = pl.pallas_call(
        _proj_body,
        grid=grid,
        in_specs=[
            pl.BlockSpec((TM, D), row),
            pl.BlockSpec((TM, D), row),
            pl.BlockSpec((TM, D), row),
            pl.BlockSpec((D, D), fixed),
            pl.BlockSpec((D, D), fixed),
            pl.BlockSpec((D, D), fixed),
            pl.BlockSpec((1, D), fixed),
            pl.BlockSpec((1, D), fixed),
            pl.BlockSpec((1, D), fixed),
        ],
        out_specs=[
            pl.BlockSpec((TM, D), row),
            pl.BlockSpec((TM, D), row),
            pl.BlockSpec((TM, D), row),
        ],
        out_shape=[jax.ShapeDtypeStruct((M, D), jnp.float32)] * 3,
    )(xq, xk, xv, Wq, Wk, Wv, bq.reshape(1, D), bk.reshape(1, D), bv.reshape(1, D))
    return out


def _corr_body(q_ref, k_ref, mv_ref):
    L = q_ref.shape[1]
    TJ = k_ref.shape[1]
    j = pl.program_id(1)
    q = q_ref[0]
    k = k_ref[0]
    g = lax.dot_general(q, k, (((1,), (1,)), ((), ())),
                        preferred_element_type=jnp.float32)
    lane = lax.broadcasted_iota(jnp.int32, (L, TJ), 1)
    x = g
    nbits = int(math.log2(TJ))
    for kb in range(nbits):
        sh = 1 << kb
        rolled = jnp.roll(x, -sh, axis=0)
        x = jnp.where((lane & sh) != 0, rolled, x)
    contrib = jnp.sum(x, axis=1, keepdims=True)
    cat = jnp.concatenate([contrib, contrib], axis=0)
    shifted = cat[pl.ds(j * TJ, L), :]
    scaled = shifted * (1.0 / q_ref.shape[2])

    @pl.when(j == 0)
    def _():
        mv_ref[0] = scaled

    @pl.when(j > 0)
    def _():
        mv_ref[0] += scaled


def _mean_corr(q3, k3):
    B, L, D = q3.shape
    TJ = 128
    grid = (B, L // TJ)
    mv = pl.pallas_call(
        _corr_body,
        grid=grid,
        in_specs=[
            pl.BlockSpec((1, L, D), lambda b, j: (b, 0, 0)),
            pl.BlockSpec((1, TJ, D), lambda b, j: (b, j, 0)),
        ],
        out_specs=pl.BlockSpec((1, L, 1), lambda b, j: (b, 0, 0)),
        out_shape=jax.ShapeDtypeStruct((B, L, 1), jnp.float32),
        compiler_params=pltpu.CompilerParams(
            dimension_semantics=("arbitrary", "arbitrary")),
    )(q3, k3)
    return mv.reshape(B, L)


def _topk_body(mv_ref, idx_ref, sw_ref, *, topk):
    mv = mv_ref[...]
    Bb, L = mv.shape
    bm = jnp.mean(mv, axis=0, keepdims=True)
    colL = lax.broadcasted_iota(jnp.int32, (1, L), 1)
    colS = lax.broadcasted_iota(jnp.int32, (1, 128), 1)
    idx_row = jnp.zeros((1, 128), jnp.int32)
    w_acc = jnp.zeros((Bb, 128), jnp.float32)
    for p in range(topk):
        mx = jnp.max(bm)
        cand = jnp.where(bm == mx, colL, jnp.int32(2**30))
        fidx = jnp.min(cand)
        idx_row = jnp.where(colS == p, fidx, idx_row)
        wcol = jnp.sum(jnp.where(colL == fidx, mv, 0.0), axis=1, keepdims=True)
        w_acc = jnp.where(colS == p, wcol, w_acc)
        bm = jnp.where(colL == fidx, -jnp.inf, bm)
    mask = colS < topk
    m = jnp.max(jnp.where(mask, w_acc, -jnp.inf), axis=1, keepdims=True)
    e = jnp.where(mask, jnp.exp(w_acc - m), 0.0)
    sw = e / jnp.sum(e, axis=1, keepdims=True)
    idx_ref[...] = idx_row
    sw_ref[...] = sw


def _topk_weights(mv, topk):
    B, L = mv.shape
    idx, sw = pl.pallas_call(
        functools.partial(_topk_body, topk=topk),
        grid=(1,),
        in_specs=[pl.BlockSpec((B, L), lambda i: (0, 0))],
        out_specs=[
            pl.BlockSpec((1, 128), lambda i: (0, 0)),
            pl.BlockSpec((B, 128), lambda i: (0, 0)),
        ],
        out_shape=[
            jax.ShapeDtypeStruct((1, 128), jnp.int32),
            jax.ShapeDtypeStruct((B, 128), jnp.float32),
        ],
    )(mv)
    return idx, sw


def _agg_body(idx_ref, sw_ref, v_ref, wo_ref, bo_ref, out_ref, *, topk):
    b = pl.program_id(0)
    v = v_ref[0]
    L = v.shape[0]
    cat = jnp.concatenate([v, v], axis=0)
    agg = sw_ref[b, 0] * cat[pl.ds(idx_ref[0, 0], L), :]
    for i in range(1, topk):
        agg += sw_ref[b, i] * cat[pl.ds(idx_ref[0, i], L), :]
    out_ref[0] = jnp.dot(agg, wo_ref[...],
                         preferred_element_type=jnp.float32) + bo_ref[...]


def _aggregate(v3, idx, sw, Wo, bo, topk):
    B, L, D = v3.shape
    out = pl.pallas_call(
        functools.partial(_agg_body, topk=topk),
        grid=(B,),
        in_specs=[
            pl.BlockSpec(memory_space=pltpu.SMEM),
            pl.BlockSpec(memory_space=pltpu.SMEM),
            pl.BlockSpec((1, L, D), lambda b: (b, 0, 0)),
            pl.BlockSpec((D, D), lambda b: (0, 0)),
            pl.BlockSpec((1, D), lambda b: (0, 0)),
        ],
        out_specs=pl.BlockSpec((1, L, D), lambda b: (b, 0, 0)),
        out_shape=jax.ShapeDtypeStruct((B, L, D), jnp.float32),
    )(idx, sw, v3, Wo, bo.reshape(1, D))
    return out


def kernel(queries, keys, values, Wq, bq, Wk, bk, Wv, bv, Wo, bo):
    B, L, D = queries.shape
    topk = int(math.log(L))
    q2, k2, v2 = _projections(
        queries.reshape(B * L, D), keys.reshape(B * L, D),
        values.reshape(B * L, D), Wq, bq, Wk, bk, Wv, bv)
    q3 = q2.reshape(B, L, D)
    k3 = k2.reshape(B, L, D)
    v3 = v2.reshape(B, L, D)
    mv = _mean_corr(q3, k3)
    idx, sw = _topk_weights(mv, topk)
    return _aggregate(v3, idx, sw, Wo, bo, topk)


# speedup vs baseline: 5.1031x; 5.1031x over previous
import functools
import math

import jax
import jax.numpy as jnp
from jax import lax
from jax.experimental import pallas as pl
from jax.experimental.pallas import tpu as pltpu


def _proj_body(xq, xk, xv, wq, wk, wv, bq, bk, bv, qo, ko, vo):
    qo[...] = jnp.dot(xq[...], wq[...], preferred_element_type=jnp.float32) + bq[...]
    ko[...] = jnp.dot(xk[...], wk[...], preferred_element_type=jnp.float32) + bk[...]
    vo[...] = jnp.dot(xv[...], wv[...], preferred_element_type=jnp.float32) + bv[...]


def _projections(xq, xk, xv, Wq, bq, Wk, bk, Wv, bv):
    M, D = xq.shape
    TM = 512
    grid = (M // TM,)
    row = lambda i: (i, 0)
    fixed = lambda i: (0, 0)
    out = pl.pallas_call(
        _proj_body,
        grid=grid,
        in_specs=[
            pl.BlockSpec((TM, D), row),
            pl.BlockSpec((TM, D), row),
            pl.BlockSpec((TM, D), row),
            pl.BlockSpec((D, D), fixed),
            pl.BlockSpec((D, D), fixed),
            pl.BlockSpec((D, D), fixed),
            pl.BlockSpec((1, D), fixed),
            pl.BlockSpec((1, D), fixed),
            pl.BlockSpec((1, D), fixed),
        ],
        out_specs=[
            pl.BlockSpec((TM, D), row),
            pl.BlockSpec((TM, D), row),
            pl.BlockSpec((TM, D), row),
        ],
        out_shape=[jax.ShapeDtypeStruct((M, D), jnp.float32)] * 3,
    )(xq, xk, xv, Wq, Wk, Wv, bq.reshape(1, D), bk.reshape(1, D), bv.reshape(1, D))
    return out


def _corr_body(q_ref, k_ref, mv_ref):
    L = q_ref.shape[1]
    TJ = k_ref.shape[1]
    j = pl.program_id(1)
    q = q_ref[0]
    k = k_ref[0]
    g = lax.dot_general(q, k, (((1,), (1,)), ((), ())),
                        preferred_element_type=jnp.float32)
    lane = lax.broadcasted_iota(jnp.int32, (L, TJ), 1)
    x = g
    nbits = int(math.log2(TJ))
    for kb in range(nbits):
        sh = 1 << kb
        rolled = jnp.roll(x, -sh, axis=0)
        x = jnp.where((lane & sh) != 0, rolled, x)
    contrib = jnp.sum(x, axis=1, keepdims=True)
    shifted = pltpu.roll(contrib, L - j * TJ, axis=0)
    scaled = shifted * (1.0 / q_ref.shape[2])

    @pl.when(j == 0)
    def _():
        mv_ref[0] = scaled

    @pl.when(j > 0)
    def _():
        mv_ref[0] += scaled


def _mean_corr(q3, k3):
    B, L, D = q3.shape
    TJ = 128
    grid = (B, L // TJ)
    mv = pl.pallas_call(
        _corr_body,
        grid=grid,
        in_specs=[
            pl.BlockSpec((1, L, D), lambda b, j: (b, 0, 0)),
            pl.BlockSpec((1, TJ, D), lambda b, j: (b, j, 0)),
        ],
        out_specs=pl.BlockSpec((1, L, 1), lambda b, j: (b, 0, 0)),
        out_shape=jax.ShapeDtypeStruct((B, L, 1), jnp.float32),
        compiler_params=pltpu.CompilerParams(
            dimension_semantics=("arbitrary", "arbitrary")),
    )(q3, k3)
    return mv.reshape(B, L)


def _topk_body(mv_ref, idx_ref, sw_ref, *, topk):
    mv = mv_ref[...]
    Bb, L = mv.shape
    bm = jnp.mean(mv, axis=0, keepdims=True)
    colL = lax.broadcasted_iota(jnp.int32, (1, L), 1)
    colS = lax.broadcasted_iota(jnp.int32, (1, 128), 1)
    idx_row = jnp.zeros((1, 128), jnp.int32)
    w_acc = jnp.zeros((Bb, 128), jnp.float32)
    for p in range(topk):
        mx = jnp.max(bm)
        cand = jnp.where(bm == mx, colL, jnp.int32(2**30))
        fidx = jnp.min(cand)
        idx_row = jnp.where(colS == p, fidx, idx_row)
        wcol = jnp.sum(jnp.where(colL == fidx, mv, 0.0), axis=1, keepdims=True)
        w_acc = jnp.where(colS == p, wcol, w_acc)
        bm = jnp.where(colL == fidx, -jnp.inf, bm)
    mask = colS < topk
    m = jnp.max(jnp.where(mask, w_acc, -jnp.inf), axis=1, keepdims=True)
    e = jnp.where(mask, jnp.exp(w_acc - m), 0.0)
    sw = e / jnp.sum(e, axis=1, keepdims=True)
    idx_ref[...] = idx_row
    sw_ref[...] = sw


def _topk_weights(mv, topk):
    B, L = mv.shape
    idx, sw = pl.pallas_call(
        functools.partial(_topk_body, topk=topk),
        grid=(1,),
        in_specs=[pl.BlockSpec((B, L), lambda i: (0, 0))],
        out_specs=[
            pl.BlockSpec((1, 128), lambda i: (0, 0)),
            pl.BlockSpec((B, 128), lambda i: (0, 0)),
        ],
        out_shape=[
            jax.ShapeDtypeStruct((1, 128), jnp.int32),
            jax.ShapeDtypeStruct((B, 128), jnp.float32),
        ],
    )(mv)
    return idx, sw


def _agg_body(idx_ref, sw_ref, v_ref, out_ref, *, topk):
    b = pl.program_id(0)
    v = v_ref[0]
    L = v.shape[0]
    out_ref[0] = sw_ref[b, 0] * pltpu.roll(v, L - idx_ref[0, 0], axis=0)
    for i in range(1, topk):
        out_ref[0] += sw_ref[b, i] * pltpu.roll(v, L - idx_ref[0, i], axis=0)


def _aggregate(v3, idx, sw, topk):
    B, L, D = v3.shape
    out = pl.pallas_call(
        functools.partial(_agg_body, topk=topk),
        grid=(B,),
        in_specs=[
            pl.BlockSpec(memory_space=pltpu.SMEM),
            pl.BlockSpec(memory_space=pltpu.SMEM),
            pl.BlockSpec((1, L, D), lambda b: (b, 0, 0)),
        ],
        out_specs=pl.BlockSpec((1, L, D), lambda b: (b, 0, 0)),
        out_shape=jax.ShapeDtypeStruct((B, L, D), jnp.float32),
    )(idx, sw, v3)
    return out


def _outproj_body(x_ref, w_ref, b_ref, o_ref):
    o_ref[...] = jnp.dot(x_ref[...], w_ref[...],
                         preferred_element_type=jnp.float32) + b_ref[...]


def _out_projection(x, Wo, bo):
    M, D = x.shape
    TM = 512
    return pl.pallas_call(
        _outproj_body,
        grid=(M // TM,),
        in_specs=[
            pl.BlockSpec((TM, D), lambda i: (i, 0)),
            pl.BlockSpec((D, D), lambda i: (0, 0)),
            pl.BlockSpec((1, D), lambda i: (0, 0)),
        ],
        out_specs=pl.BlockSpec((TM, D), lambda i: (i, 0)),
        out_shape=jax.ShapeDtypeStruct((M, D), jnp.float32),
    )(x, Wo, bo.reshape(1, D))


def kernel(queries, keys, values, Wq, bq, Wk, bk, Wv, bv, Wo, bo):
    B, L, D = queries.shape
    topk = int(math.log(L))
    q2, k2, v2 = _projections(
        queries.reshape(B * L, D), keys.reshape(B * L, D),
        values.reshape(B * L, D), Wq, bq, Wk, bk, Wv, bv)
    q3 = q2.reshape(B, L, D)
    k3 = k2.reshape(B, L, D)
    v3 = v2.reshape(B, L, D)
    mv = _mean_corr(q3, k3)
    idx, sw = _topk_weights(mv, topk)
    agg = _aggregate(v3, idx, sw, topk)
    out = _out_projection(agg.reshape(B * L, D), Wo, bo)
    return out.reshape(B, L, D)
